# baseline (speedup 1.0000x reference)
import functools

import jax
import jax.numpy as jnp
from jax import lax
from jax.experimental import pallas as pl
from jax.experimental.pallas import tpu as pltpu
from jax.experimental.pallas import tpu_sc as plsc

_NUM_CORES = 2
_NUM_SUBCORES = 16
_NW = _NUM_CORES * _NUM_SUBCORES
_LANES = 16
_PAD_D = 128


def _make_pool(ctx, b, d):
    rows_per_w = b // _NW
    n_cvec = d // _LANES
    mesh = plsc.VectorSubcoreMesh(core_axis_name="c", subcore_axis_name="s")

    @functools.partial(
        pl.kernel,
        out_type=jax.ShapeDtypeStruct((b * d,), jnp.float32),
        mesh=mesh,
        scratch_types=[
            pltpu.VMEM((rows_per_w,), jnp.int32),
            pltpu.VMEM((rows_per_w,), jnp.int32),
            pltpu.VMEM((rows_per_w, _PAD_D), jnp.float32),
            pltpu.VMEM((rows_per_w, _PAD_D), jnp.float32),
            pltpu.VMEM((rows_per_w * d,), jnp.float32),
            pltpu.SemaphoreType.DMA,
            pltpu.SemaphoreType.DMA,
        ],
    )
    def pool(xf_hbm, table_hbm, h_hbm, idx0, idx1, rows0, rows1, acc_v, s0, s1):
        wid = lax.axis_index("s") * _NUM_CORES + lax.axis_index("c")
        base = wid * rows_per_w
        idx = (idx0, idx1)
        rows = (rows0, rows1)
        sems = (s0, s1)

        def start(j):
            k = j % 2
            pltpu.sync_copy(xf_hbm.at[pl.ds(j * b + base, rows_per_w)], idx[k])
            return pltpu.async_copy(table_hbm.at[idx[k]], rows[k], sems[k])

        handles = [start(0), start(1)]
        for j in range(ctx):
            handles[j % 2].wait()
            rv = rows[j % 2]

            if j == 0:

                def init_row(r, carry):
                    for c in range(n_cvec):
                        acc_v[pl.ds(r * d + c * _LANES, _LANES)] = rv[
                            r, pl.ds(c * _LANES, _LANES)
                        ]
                    return carry

                lax.fori_loop(0, rows_per_w, init_row, 0)
            else:

                def add_row(r, carry):
                    for c in range(n_cvec):
                        acc_v[pl.ds(r * d + c * _LANES, _LANES)] += rv[
                            r, pl.ds(c * _LANES, _LANES)
                        ]
                    return carry

                lax.fori_loop(0, rows_per_w, add_row, 0)

            if j + 2 < ctx:
                handles[j % 2] = start(j + 2)

        pltpu.sync_copy(acc_v, h_hbm.at[pl.ds(base * d, rows_per_w * d)])

    return pool


def _matmul_body(scale, h_ref, w_ref, b_ref, out_ref):
    h = h_ref[...] * scale
    out_ref[...] = (
        lax.dot_general(
            h,
            w_ref[...],
            dimension_numbers=(((1,), (1,)), ((), ())),
            preferred_element_type=jnp.float32,
        )
        + b_ref[...]
    )


def _projection(h_sum, w, b2d, ctx, bb, vb):
    batch, d = h_sum.shape
    vocab = w.shape[0]
    nb = batch // bb
    nv = pl.cdiv(vocab, vb)
    return pl.pallas_call(
        functools.partial(_matmul_body, float(1.0 / ctx)),
        grid=(nv, nb),
        in_specs=[
            pl.BlockSpec((bb, d), lambda j, i: (i, 0)),
            pl.BlockSpec((vb, d), lambda j, i: (j, 0)),
            pl.BlockSpec((1, vb), lambda j, i: (0, j)),
        ],
        out_specs=pl.BlockSpec((bb, vb), lambda j, i: (i, j)),
        out_shape=jax.ShapeDtypeStruct((batch, vocab), jnp.float32),
        compiler_params=pltpu.CompilerParams(
            dimension_semantics=("arbitrary", "arbitrary"),
        ),
    )(h_sum, w, b2d)


def kernel(x, emb_table, W, b):
    batch, ctx = x.shape
    vocab, d = W.shape
    x_flat = x.T.reshape(-1)
    table128 = jnp.pad(emb_table, ((0, 0), (0, _PAD_D - d)))
    h_sum = _make_pool(ctx, batch, d)(x_flat, table128).reshape(batch, d)
    return _projection(h_sum, W, b.reshape(1, vocab), ctx, 512, 2048)


# speedup vs baseline: 3.2265x; 3.2265x over previous
import functools

import jax
import jax.numpy as jnp
from jax import lax
from jax.experimental import pallas as pl
from jax.experimental.pallas import tpu as pltpu
from jax.experimental.pallas import tpu_sc as plsc

_NUM_CORES = 2
_NUM_SUBCORES = 16
_NW = _NUM_CORES * _NUM_SUBCORES
_LANES = 16


def _make_pool(ctx, b, d):
    rows_per_w = b // _NW
    n_cvec = d // _LANES
    mesh = plsc.VectorSubcoreMesh(core_axis_name="c", subcore_axis_name="s")

    @functools.partial(
        pl.kernel,
        out_type=jax.ShapeDtypeStruct((b * d,), jnp.float32),
        mesh=mesh,
        scratch_types=[
            pltpu.VMEM((rows_per_w,), jnp.int32),
            pltpu.VMEM((rows_per_w,), jnp.int32),
            pltpu.VMEM((rows_per_w, d), jnp.float32),
            pltpu.VMEM((rows_per_w, d), jnp.float32),
            pltpu.VMEM((rows_per_w * d,), jnp.float32),
            pltpu.SemaphoreType.DMA,
            pltpu.SemaphoreType.DMA,
        ],
        compiler_params=pltpu.CompilerParams(use_tc_tiling_on_sc=False),
    )
    def pool(xf_hbm, table_hbm, h_hbm, idx0, idx1, rows0, rows1, acc_v, s0, s1):
        wid = lax.axis_index("s") * _NUM_CORES + lax.axis_index("c")
        base = wid * rows_per_w
        idx = (idx0, idx1)
        rows = (rows0, rows1)
        sems = (s0, s1)

        def start(j):
            k = j % 2
            pltpu.sync_copy(xf_hbm.at[pl.ds(j * b + base, rows_per_w)], idx[k])
            return pltpu.async_copy(table_hbm.at[idx[k]], rows[k], sems[k])

        handles = [start(0), start(1)]
        for j in range(ctx):
            handles[j % 2].wait()
            rv = rows[j % 2]

            if j == 0:

                def init_row(r, carry):
                    for c in range(n_cvec):
                        acc_v[pl.ds(r * d + c * _LANES, _LANES)] = rv[
                            r, pl.ds(c * _LANES, _LANES)
                        ]
                    return carry

                lax.fori_loop(0, rows_per_w, init_row, 0)
            else:

                def add_row(r, carry):
                    for c in range(n_cvec):
                        acc_v[pl.ds(r * d + c * _LANES, _LANES)] += rv[
                            r, pl.ds(c * _LANES, _LANES)
                        ]
                    return carry

                lax.fori_loop(0, rows_per_w, add_row, 0)

            if j + 2 < ctx:
                handles[j % 2] = start(j + 2)

        pltpu.sync_copy(acc_v, h_hbm.at[pl.ds(base * d, rows_per_w * d)])

    return pool


def _matmul_body(scale, h_ref, wt_ref, b_ref, out_ref):
    h = h_ref[...] * scale
    acc = lax.dot_general(
        wt_ref[...],
        h,
        dimension_numbers=(((0,), (1,)), ((), ())),
        preferred_element_type=jnp.float32,
    )
    out_ref[...] = acc + jnp.transpose(b_ref[...])


def _projection(h_sum, wt, b2d, ctx, vb):
    batch, d = h_sum.shape
    vocab = wt.shape[1]
    nv = pl.cdiv(vocab, vb)
    out_t = pl.pallas_call(
        functools.partial(_matmul_body, float(1.0 / ctx)),
        grid=(nv,),
        in_specs=[
            pl.BlockSpec((batch, d), lambda j: (0, 0)),
            pl.BlockSpec((d, vb), lambda j: (0, j)),
            pl.BlockSpec((1, vb), lambda j: (0, j)),
        ],
        out_specs=pl.BlockSpec((vb, batch), lambda j: (j, 0)),
        out_shape=jax.ShapeDtypeStruct((vocab, batch), jnp.float32),
        compiler_params=pltpu.CompilerParams(
            dimension_semantics=("arbitrary",),
        ),
    )(h_sum, wt, b2d)
    return out_t.T


def kernel(x, emb_table, W, b):
    batch, ctx = x.shape
    vocab, d = W.shape
    x_flat = x.T.reshape(-1)
    h_sum = _make_pool(ctx, batch, d)(x_flat, emb_table).reshape(batch, d)
    return _projection(h_sum, W.T, b.reshape(1, vocab), ctx, 256)


# speedup vs baseline: 3.6404x; 1.1283x over previous
import functools

import jax
import jax.numpy as jnp
from jax import lax
from jax.experimental import pallas as pl
from jax.experimental.pallas import tpu as pltpu
from jax.experimental.pallas import tpu_sc as plsc

_NUM_CORES = 2
_NUM_SUBCORES = 16
_NW = _NUM_CORES * _NUM_SUBCORES
_LANES = 16


def _make_pool(ctx, b, d):
    rows_per_w = b // _NW
    n_cvec = d // _LANES
    mesh = plsc.VectorSubcoreMesh(core_axis_name="c", subcore_axis_name="s")

    @functools.partial(
        pl.kernel,
        out_type=jax.ShapeDtypeStruct((b * d,), jnp.float32),
        mesh=mesh,
        scratch_types=[
            pltpu.VMEM((rows_per_w,), jnp.int32),
            pltpu.VMEM((rows_per_w,), jnp.int32),
            pltpu.VMEM((rows_per_w, d), jnp.float32),
            pltpu.VMEM((rows_per_w, d), jnp.float32),
            pltpu.VMEM((rows_per_w * d,), jnp.float32),
            pltpu.SemaphoreType.DMA,
            pltpu.SemaphoreType.DMA,
        ],
        compiler_params=pltpu.CompilerParams(use_tc_tiling_on_sc=False),
    )
    def pool(xf_hbm, table_hbm, h_hbm, idx0, idx1, rows0, rows1, acc_v, s0, s1):
        wid = lax.axis_index("s") * _NUM_CORES + lax.axis_index("c")
        base = wid * rows_per_w
        idx = (idx0, idx1)
        rows = (rows0, rows1)
        sems = (s0, s1)

        def start(j):
            k = j % 2
            pltpu.sync_copy(xf_hbm.at[pl.ds(j * b + base, rows_per_w)], idx[k])
            return pltpu.async_copy(table_hbm.at[idx[k]], rows[k], sems[k])

        handles = [start(0), start(1)]
        for j in range(ctx):
            handles[j % 2].wait()
            rv = rows[j % 2]

            if j == 0:

                def init_row(r, carry):
                    for c in range(n_cvec):
                        acc_v[pl.ds(r * d + c * _LANES, _LANES)] = rv[
                            r, pl.ds(c * _LANES, _LANES)
                        ]
                    return carry

                lax.fori_loop(0, rows_per_w, init_row, 0)
            else:

                def add_row(r, carry):
                    for c in range(n_cvec):
                        acc_v[pl.ds(r * d + c * _LANES, _LANES)] += rv[
                            r, pl.ds(c * _LANES, _LANES)
                        ]
                    return carry

                lax.fori_loop(0, rows_per_w, add_row, 0)

            if j + 2 < ctx:
                handles[j % 2] = start(j + 2)

        pltpu.sync_copy(acc_v, h_hbm.at[pl.ds(base * d, rows_per_w * d)])

    return pool


def _matmul_body(scale, h_ref, wt_ref, b_ref, out_ref):
    h = h_ref[...] * scale
    acc = lax.dot_general(
        wt_ref[...],
        h,
        dimension_numbers=(((0,), (1,)), ((), ())),
        preferred_element_type=jnp.float32,
    )
    out_ref[...] = acc + jnp.transpose(b_ref[...])


def _projection(h_sum, wt, b2d, ctx, vb):
    batch, d = h_sum.shape
    vocab = wt.shape[1]
    nv = pl.cdiv(vocab, vb)
    out_t = pl.pallas_call(
        functools.partial(_matmul_body, float(1.0 / ctx)),
        grid=(nv,),
        in_specs=[
            pl.BlockSpec((batch, d), lambda j: (0, 0)),
            pl.BlockSpec((d, vb), lambda j: (0, j)),
            pl.BlockSpec((1, vb), lambda j: (0, j)),
        ],
        out_specs=pl.BlockSpec((vb, batch), lambda j: (j, 0)),
        out_shape=jax.ShapeDtypeStruct((vocab, batch), jnp.float32),
        compiler_params=pltpu.CompilerParams(
            dimension_semantics=("arbitrary",),
        ),
    )(h_sum, wt, b2d)
    return out_t.T


def kernel(x, emb_table, W, b):
    batch, ctx = x.shape
    vocab, d = W.shape
    x_flat = x.T.reshape(-1)
    h_sum = _make_pool(ctx, batch, d)(x_flat, emb_table).reshape(batch, d)
    return _projection(h_sum, W.T, b.reshape(1, vocab), ctx, 512)


# speedup vs baseline: 3.6550x; 1.0040x over previous
import functools

import jax
import jax.numpy as jnp
from jax import lax
from jax.experimental import pallas as pl
from jax.experimental.pallas import tpu as pltpu
from jax.experimental.pallas import tpu_sc as plsc

_NUM_CORES = 2
_NUM_SUBCORES = 16
_NW = _NUM_CORES * _NUM_SUBCORES
_LANES = 16


def _make_pool(ctx, b, d):
    rows_per_w = b // _NW
    n_cvec = d // _LANES
    mesh = plsc.VectorSubcoreMesh(core_axis_name="c", subcore_axis_name="s")

    @functools.partial(
        pl.kernel,
        out_type=jax.ShapeDtypeStruct((b * d,), jnp.float32),
        mesh=mesh,
        scratch_types=[
            pltpu.VMEM((rows_per_w,), jnp.int32),
            pltpu.VMEM((rows_per_w,), jnp.int32),
            pltpu.VMEM((rows_per_w, d), jnp.float32),
            pltpu.VMEM((rows_per_w, d), jnp.float32),
            pltpu.VMEM((rows_per_w * d,), jnp.float32),
            pltpu.SemaphoreType.DMA,
            pltpu.SemaphoreType.DMA,
        ],
        compiler_params=pltpu.CompilerParams(use_tc_tiling_on_sc=False),
    )
    def pool(xf_hbm, table_hbm, h_hbm, idx0, idx1, rows0, rows1, acc_v, s0, s1):
        wid = lax.axis_index("s") * _NUM_CORES + lax.axis_index("c")
        base = wid * rows_per_w
        idx = (idx0, idx1)
        rows = (rows0, rows1)
        sems = (s0, s1)

        def start(j):
            k = j % 2
            pltpu.sync_copy(xf_hbm.at[pl.ds(j * b + base, rows_per_w)], idx[k])
            return pltpu.async_copy(table_hbm.at[idx[k]], rows[k], sems[k])

        handles = [start(0), start(1)]
        for j in range(ctx):
            handles[j % 2].wait()
            rv = rows[j % 2]

            if j == 0:

                def init_row(r, carry):
                    for c in range(n_cvec):
                        acc_v[pl.ds(r * d + c * _LANES, _LANES)] = rv[
                            r, pl.ds(c * _LANES, _LANES)
                        ]
                    return carry

                lax.fori_loop(0, rows_per_w, init_row, 0)
            else:

                def add_row(r, carry):
                    for c in range(n_cvec):
                        acc_v[pl.ds(r * d + c * _LANES, _LANES)] += rv[
                            r, pl.ds(c * _LANES, _LANES)
                        ]
                    return carry

                lax.fori_loop(0, rows_per_w, add_row, 0)

            if j + 2 < ctx:
                handles[j % 2] = start(j + 2)

        pltpu.sync_copy(acc_v, h_hbm.at[pl.ds(base * d, rows_per_w * d)])

    return pool


def _matmul_body(scale, h_ref, wt_ref, b_ref, out_ref):
    h = h_ref[...] * scale
    acc = lax.dot_general(
        wt_ref[...],
        h,
        dimension_numbers=(((0,), (1,)), ((), ())),
        preferred_element_type=jnp.float32,
    )
    out_ref[...] = acc + jnp.transpose(b_ref[...])


def _projection(h_sum, wt, b2d, ctx, vb):
    batch, d = h_sum.shape
    vocab = wt.shape[1]
    nv = pl.cdiv(vocab, vb)
    out_t = pl.pallas_call(
        functools.partial(_matmul_body, float(1.0 / ctx)),
        grid=(nv,),
        in_specs=[
            pl.BlockSpec((batch, d), lambda j: (0, 0)),
            pl.BlockSpec((d, vb), lambda j: (0, j)),
            pl.BlockSpec((1, vb), lambda j: (0, j)),
        ],
        out_specs=pl.BlockSpec((vb, batch), lambda j: (j, 0)),
        out_shape=jax.ShapeDtypeStruct((vocab, batch), jnp.float32),
        compiler_params=pltpu.CompilerParams(
            dimension_semantics=("arbitrary",),
        ),
    )(h_sum, wt, b2d)
    return out_t.T


def kernel(x, emb_table, W, b):
    batch, ctx = x.shape
    vocab, d = W.shape
    x_flat = x.T.reshape(-1)
    h_sum = _make_pool(ctx, batch, d)(x_flat, emb_table).reshape(batch, d)
    return _projection(h_sum, W.T, b.reshape(1, vocab), ctx, 1024)
